# baseline (speedup 1.0000x reference)
import functools

import jax
import jax.numpy as jnp
from jax import lax
from jax.experimental import pallas as pl
from jax.experimental.pallas import tpu as pltpu
from jax.experimental.pallas import tpu_sc as plsc

ONEHOT = 38
BATCH = 16384

_INFO = plsc.get_sparse_core_info()
NC = _INFO.num_cores
NS = _INFO.num_subcores
LANES = _INFO.num_lanes
NW = NC * NS
CPW = BATCH // NW
NCHUNK = 2
CCOLS = CPW // NCHUNK

_mesh = plsc.VectorSubcoreMesh(core_axis_name="c", subcore_axis_name="s")


@functools.partial(
    pl.kernel,
    mesh=_mesh,
    out_type=jax.ShapeDtypeStruct((ONEHOT, BATCH), jnp.float32),
    scratch_types=[
        pltpu.VMEM((CPW,), jnp.int32),
        pltpu.VMEM((ONEHOT, CPW), jnp.float32),
        pltpu.SemaphoreType.DMA,
    ],
    compiler_params=pltpu.CompilerParams(needs_layout_passes=False),
)
def _onehot_sc(idx_hbm, out_hbm, idx_v, blk_v, sem):
    wid = lax.axis_index("s") * NC + lax.axis_index("c")
    base = wid * CPW

    pltpu.sync_copy(idx_hbm.at[pl.ds(base, CPW)], idx_v)

    zeros = jnp.zeros((LANES,), jnp.float32)
    ones = jnp.ones((LANES,), jnp.float32)
    lane = lax.iota(jnp.int32, LANES)

    copies = []
    for k in range(NCHUNK):
        def zero_body(r, carry, k=k):
            for c in range(CCOLS // LANES):
                blk_v[r, pl.ds(k * CCOLS + c * LANES, LANES)] = zeros
            return carry

        lax.fori_loop(0, ONEHOT, zero_body, 0)

        def scatter_body(g, carry, k=k):
            c0 = k * CCOLS + g * LANES
            rows = idx_v[pl.ds(c0, LANES)]
            cols = c0 + lane
            plsc.store_scatter(blk_v, [rows, cols], ones)
            return carry

        lax.fori_loop(0, CCOLS // LANES, scatter_body, 0)

        copies.append(
            pltpu.async_copy(
                blk_v.at[:, pl.ds(k * CCOLS, CCOLS)],
                out_hbm.at[:, pl.ds(base + k * CCOLS, CCOLS)],
                sem,
            )
        )
    for c in copies:
        c.wait()


def kernel(input_char):
    return _onehot_sc(input_char.astype(jnp.int32)).T


# speedup vs baseline: 1.0082x; 1.0082x over previous
import functools

import jax
import jax.numpy as jnp
from jax import lax
from jax.experimental import pallas as pl
from jax.experimental.pallas import tpu as pltpu
from jax.experimental.pallas import tpu_sc as plsc

ONEHOT = 38
BATCH = 16384

_INFO = plsc.get_sparse_core_info()
NC = 1
NS = _INFO.num_subcores
LANES = _INFO.num_lanes
NW = NC * NS
CPW = BATCH // NW
NCHUNK = 4
CCOLS = CPW // NCHUNK

_mesh = plsc.VectorSubcoreMesh(core_axis_name="c", subcore_axis_name="s", num_cores=1)


@functools.partial(
    pl.kernel,
    mesh=_mesh,
    out_type=jax.ShapeDtypeStruct((ONEHOT, BATCH), jnp.float32),
    scratch_types=[
        pltpu.VMEM((CPW,), jnp.int32),
        pltpu.VMEM((ONEHOT, CPW), jnp.float32),
        pltpu.SemaphoreType.DMA,
    ],
    compiler_params=pltpu.CompilerParams(needs_layout_passes=False),
)
def _onehot_sc(idx_hbm, out_hbm, idx_v, blk_v, sem):
    wid = lax.axis_index("s") * NC + lax.axis_index("c")
    base = wid * CPW

    pltpu.sync_copy(idx_hbm.at[pl.ds(base, CPW)], idx_v)

    zeros = jnp.zeros((LANES,), jnp.float32)
    ones = jnp.ones((LANES,), jnp.float32)
    lane = lax.iota(jnp.int32, LANES)

    copies = []
    for k in range(NCHUNK):
        def zero_body(r, carry, k=k):
            for c in range(CCOLS // LANES):
                blk_v[r, pl.ds(k * CCOLS + c * LANES, LANES)] = zeros
            return carry

        lax.fori_loop(0, ONEHOT, zero_body, 0)

        def scatter_body(g, carry, k=k):
            c0 = k * CCOLS + g * LANES
            rows = idx_v[pl.ds(c0, LANES)]
            cols = c0 + lane
            plsc.store_scatter(blk_v, [rows, cols], ones)
            return carry

        lax.fori_loop(0, CCOLS // LANES, scatter_body, 0)

        copies.append(
            pltpu.async_copy(
                blk_v.at[:, pl.ds(k * CCOLS, CCOLS)],
                out_hbm.at[:, pl.ds(base + k * CCOLS, CCOLS)],
                sem,
            )
        )
    for c in copies:
        c.wait()


def kernel(input_char):
    return _onehot_sc(input_char.astype(jnp.int32)).T
